# baseline (speedup 1.0000x reference)
import jax
import jax.numpy as jnp
from jax.experimental import pallas as pl


def _bcast_kernel(w_ref, out_ref):
    out_ref[:] = jnp.broadcast_to(w_ref[:][:, None, :], out_ref.shape)


def kernel(mask, embed_weight):
    l, d = embed_weight.shape
    b = mask.shape[0]
    rows = 8
    return pl.pallas_call(
        _bcast_kernel,
        grid=(l // rows,),
        in_specs=[pl.BlockSpec((rows, d), lambda i: (i, 0))],
        out_specs=pl.BlockSpec((rows, b, d), lambda i: (i, 0, 0)),
        out_shape=jax.ShapeDtypeStruct((l, b, d), embed_weight.dtype),
    )(embed_weight)


# speedup vs baseline: 1.0471x; 1.0471x over previous
import jax
import jax.numpy as jnp
from jax.experimental import pallas as pl


def _bcast_kernel(w_ref, out_ref):
    out_ref[:] = jnp.broadcast_to(w_ref[:][:, None, :], out_ref.shape)


def kernel(mask, embed_weight):
    l, d = embed_weight.shape
    b = mask.shape[0]
    rows = 8
    bt = 256
    return pl.pallas_call(
        _bcast_kernel,
        grid=(l // rows, b // bt),
        in_specs=[pl.BlockSpec((rows, d), lambda i, j: (i, 0))],
        out_specs=pl.BlockSpec((rows, bt, d), lambda i, j: (i, j, 0)),
        out_shape=jax.ShapeDtypeStruct((l, b, d), embed_weight.dtype),
    )(embed_weight)
